# Initial kernel scaffold; baseline (speedup 1.0000x reference)
import jax
import jax.numpy as jnp
from jax.experimental import pallas as pl


def kernel(x, edge_index, laplacian, W1, b1, W2, b2, W3, b3, W4, b4):
    raise NotImplementedError("write your pallas kernel here")


# speedup vs baseline: 2.9433x; 2.9433x over previous
import functools

import jax
import jax.numpy as jnp
from jax import lax
from jax.experimental import pallas as pl
from jax.experimental.pallas import tpu as pltpu
from jax.experimental.pallas import tpu_sc as plsc

N_NODES = 10000
N_PAD = 10240
N_EDGES = 320000
CHUNK = 128
N_CHUNKS = N_EDGES // CHUNK
K_CHEB = 6
NC = 2
NS = 16
ROWS_PER_TILE = N_PAD // NS
WB = 320


def _mesh():
    return plsc.VectorSubcoreMesh(core_axis_name="c", subcore_axis_name="s")


def _split16(sid, total):
    base = total // NS
    rem = total - base * NS
    start = sid * base + jnp.minimum(sid, rem)
    cnt = base + (sid < rem).astype(jnp.int32)
    return start, cnt


def _split32(wid, total):
    base = total // (NS * NC)
    rem = total - base * NS * NC
    start = wid * base + jnp.minimum(wid, rem)
    cnt = base + (wid < rem).astype(jnp.int32)
    return start, cnt


@functools.partial(
    pl.kernel,
    out_type=jax.ShapeDtypeStruct((NC, N_PAD), jnp.float32),
    mesh=_mesh(),
    compiler_params=pltpu.CompilerParams(needs_layout_passes=False, use_tc_tiling_on_sc=False),
    scratch_types=[
        pltpu.VMEM_SHARED((N_PAD,), jnp.float32),
        pltpu.VMEM((ROWS_PER_TILE,), jnp.float32),
        pltpu.VMEM((1, CHUNK), jnp.int32),
        pltpu.VMEM((CHUNK,), jnp.float32),
        pltpu.VMEM((ROWS_PER_TILE,), jnp.float32),
    ],
)
def _deg_kernel(row2d, lap2d, deg_out, deg_acc, zbuf, ridx, lbuf, dslice):
    cid = lax.axis_index("c")
    sid = lax.axis_index("s")
    r0 = sid * ROWS_PER_TILE

    def _zb(i, _):
        zbuf[pl.ds(i * 16, 16)] = jnp.zeros((16,), jnp.float32)
        return 0
    lax.fori_loop(0, ROWS_PER_TILE // 16, _zb, 0)
    pltpu.sync_copy(zbuf, deg_acc.at[pl.ds(r0, ROWS_PER_TILE)])
    plsc.subcore_barrier()

    wid = sid * NC + cid
    start, cnt = _split32(wid, N_CHUNKS)

    def _deg(j, _):
        gj = start + j
        pltpu.sync_copy(row2d.at[gj], ridx.at[0])
        pltpu.sync_copy(lap2d.at[gj], lbuf)
        pltpu.sync_copy(lbuf, deg_acc.at[ridx.at[0]], add=True)
        return 0
    lax.fori_loop(0, cnt, _deg, 0)
    plsc.subcore_barrier()

    pltpu.sync_copy(deg_acc.at[pl.ds(r0, ROWS_PER_TILE)], dslice)
    pltpu.sync_copy(dslice, deg_out.at[cid, pl.ds(r0, ROWS_PER_TILE)])


def _dis_body(p_ref, o_ref):
    deg = p_ref[0] + p_ref[1]
    y = lax.rsqrt(jnp.maximum(deg, 1e-12))
    o_ref[...] = jnp.where(deg > 0, y, 0.0)


_dis_kernel = pl.pallas_call(
    _dis_body,
    out_shape=jax.ShapeDtypeStruct((N_PAD // 128, 128), jnp.float32),
)


@functools.partial(
    pl.kernel,
    out_type=jax.ShapeDtypeStruct((N_CHUNKS, CHUNK), jnp.float32),
    mesh=_mesh(),
    compiler_params=pltpu.CompilerParams(needs_layout_passes=False, use_tc_tiling_on_sc=False),
    scratch_types=[
        pltpu.VMEM((1, CHUNK), jnp.int32),
        pltpu.VMEM((1, CHUNK), jnp.int32),
        pltpu.VMEM((CHUNK,), jnp.float32),
        pltpu.VMEM((CHUNK,), jnp.float32),
        pltpu.VMEM((N_PAD,), jnp.float32),
    ],
)
def _wn_kernel(row2d, col2d, lap2d, dis, wn2d, ridx, cidx, lbuf, wbuf, disbuf):
    cid = lax.axis_index("c")
    sid = lax.axis_index("s")
    pltpu.sync_copy(dis, disbuf)
    wid = sid * NC + cid
    start, cnt = _split32(wid, N_CHUNKS)

    def _wn(j, _):
        gj = start + j
        pltpu.sync_copy(row2d.at[gj], ridx.at[0])
        pltpu.sync_copy(col2d.at[gj], cidx.at[0])
        pltpu.sync_copy(lap2d.at[gj], lbuf)
        for i in range(CHUNK // 16):
            r16 = ridx[0, pl.ds(i * 16, 16)]
            c16 = cidx[0, pl.ds(i * 16, 16)]
            dr = plsc.load_gather(disbuf, [r16])
            dc = plsc.load_gather(disbuf, [c16])
            l16 = lbuf[pl.ds(i * 16, 16)]
            wbuf[pl.ds(i * 16, 16)] = -(dr * l16 * dc)
        pltpu.sync_copy(wbuf, wn2d.at[gj])
        return 0
    lax.fori_loop(0, cnt, _wn, 0)


def _make_prop_kernel(d2):
    nvec = d2 // 16

    @functools.partial(
        pl.kernel,
        out_type=jax.ShapeDtypeStruct((5, NC, N_PAD, d2), jnp.float32),
        mesh=_mesh(),
        compiler_params=pltpu.CompilerParams(needs_layout_passes=False, use_tc_tiling_on_sc=False),
        scratch_types=[
            pltpu.VMEM_SHARED((N_PAD, d2), jnp.float32),
            pltpu.VMEM((WB, d2), jnp.float32),
            pltpu.VMEM((CHUNK, d2), jnp.float32),
            pltpu.VMEM((1, CHUNK), jnp.int32),
            pltpu.VMEM((1, CHUNK), jnp.int32),
            pltpu.VMEM((CHUNK,), jnp.float32),
            pltpu.VMEM((WB, d2), jnp.float32),
            pltpu.VMEM((WB, d2), jnp.float32),
        ],
    )
    def prop_kernel(h2, row2d, col2d, wn2d, tx,
                    acc, zbuf, gbuf, ridx, cidx, wbuf, pbuf, sbuf):
        cid = lax.axis_index("c")
        sid = lax.axis_index("s")
        r0 = sid * ROWS_PER_TILE

        def _zb(i, _):
            for i2 in range(nvec):
                zbuf[i, pl.ds(i2 * 16, 16)] = jnp.zeros((16,), jnp.float32)
            return 0
        lax.fori_loop(0, WB, _zb, 0)

        start, cnt = _split16(sid, N_CHUNKS)

        for k in range(1, 6):
            pltpu.sync_copy(zbuf, acc.at[pl.ds(r0, WB)])
            pltpu.sync_copy(zbuf, acc.at[pl.ds(r0 + WB, WB)])
            plsc.subcore_barrier()

            src = h2.at[cid] if k == 1 else tx.at[k - 2, cid]

            def _edge(j, _):
                gj = start + j
                pltpu.sync_copy(row2d.at[gj], ridx.at[0])
                pltpu.sync_copy(col2d.at[gj], cidx.at[0])
                pltpu.sync_copy(wn2d.at[gj], wbuf)
                pltpu.sync_copy(src.at[cidx.at[0]], gbuf)

                def _scale(e, _2):
                    w = plsc.load_gather(wbuf, [jnp.full((16,), e, jnp.int32)])
                    for i in range(nvec):
                        gbuf[e, pl.ds(i * 16, 16)] = gbuf[e, pl.ds(i * 16, 16)] * w
                    return 0
                lax.fori_loop(0, CHUNK, _scale, 0)
                pltpu.sync_copy(gbuf, acc.at[ridx.at[0]], add=True)
                return 0
            lax.fori_loop(0, cnt, _edge, 0)
            plsc.subcore_barrier()

            for half in range(2):
                rb = r0 + half * WB
                pltpu.sync_copy(acc.at[pl.ds(rb, WB)], pbuf)
                if k > 1:
                    subsrc = h2.at[cid] if k == 2 else tx.at[k - 3, cid]
                    pltpu.sync_copy(subsrc.at[pl.ds(rb, WB)], sbuf)

                    def _fix(r, _2):
                        for i in range(nvec):
                            pbuf[r, pl.ds(i * 16, 16)] = (
                                2.0 * pbuf[r, pl.ds(i * 16, 16)]
                                - sbuf[r, pl.ds(i * 16, 16)])
                        return 0
                    lax.fori_loop(0, WB, _fix, 0)
                pltpu.sync_copy(pbuf, tx.at[k - 1, cid].at[pl.ds(rb, WB)])
            plsc.subcore_barrier()

    return prop_kernel


def _make_mm_kernel(din, dout):
    d2i, d2o = din // 2, dout // 2
    bn = 1024

    def mm(h_ref, tx_ref, w_ref, b_ref, o_ref):
        acc = jnp.broadcast_to(b_ref[0], (bn, dout))
        for c in range(2):
            acc = acc + jnp.dot(h_ref[c], w_ref[0, c * d2i:(c + 1) * d2i, :],
                                preferred_element_type=jnp.float32)
            for k in range(1, K_CHEB):
                acc = acc + jnp.dot(tx_ref[k - 1, c],
                                    w_ref[k, c * d2i:(c + 1) * d2i, :],
                                    preferred_element_type=jnp.float32)
        acc = jnp.maximum(acc, 0.0)
        for c in range(2):
            o_ref[c] = acc[:, c * d2o:(c + 1) * d2o]

    return pl.pallas_call(
        mm,
        grid=(N_PAD // bn,),
        in_specs=[
            pl.BlockSpec((2, bn, d2i), lambda i: (0, i, 0)),
            pl.BlockSpec((5, 2, bn, d2i), lambda i: (0, 0, i, 0)),
            pl.BlockSpec((K_CHEB, din, dout), lambda i: (0, 0, 0)),
            pl.BlockSpec((1, dout), lambda i: (0, 0)),
        ],
        out_specs=pl.BlockSpec((2, bn, d2o), lambda i: (0, i, 0)),
        out_shape=jax.ShapeDtypeStruct((2, N_PAD, d2o), jnp.float32),
    )


_PROP = {128: _make_prop_kernel(64), 64: _make_prop_kernel(32),
         32: _make_prop_kernel(16)}
_MM = {(128, 64): _make_mm_kernel(128, 64), (64, 32): _make_mm_kernel(64, 32),
       (32, 64): _make_mm_kernel(32, 64), (64, 128): _make_mm_kernel(64, 128)}


def kernel(x, edge_index, laplacian, W1, b1, W2, b2, W3, b3, W4, b4):
    row2d = edge_index[0].reshape(N_CHUNKS, CHUNK)
    col2d = edge_index[1].reshape(N_CHUNKS, CHUNK)
    lap2d = laplacian.reshape(N_CHUNKS, CHUNK)

    deg_p = _deg_kernel(row2d, lap2d)
    dis = _dis_kernel(deg_p.reshape(NC, N_PAD // 128, 128)).reshape(N_PAD)
    wn2d = _wn_kernel(row2d, col2d, lap2d, dis)

    xp = jnp.pad(x, ((0, N_PAD - N_NODES), (0, 0)))
    h = xp.reshape(N_PAD, 2, 64).transpose(1, 0, 2)

    layers = [(128, 64, W1, b1), (64, 32, W2, b2),
              (32, 64, W3, b3), (64, 128, W4, b4)]
    for din, dout, W, b in layers:
        tx = _PROP[din](h, row2d, col2d, wn2d)
        h = _MM[(din, dout)](h, tx, W, b.reshape(1, dout))

    return jnp.concatenate([h[0, :N_NODES], h[1, :N_NODES]], axis=1)


# speedup vs baseline: 6.8998x; 2.3443x over previous
import functools

import jax
import jax.numpy as jnp
from jax import lax
from jax.experimental import pallas as pl
from jax.experimental.pallas import tpu as pltpu
from jax.experimental.pallas import tpu_sc as plsc

N_NODES = 10000
N_PAD = 10240
N_EDGES = 320000
CHUNK = 128
N_CHUNKS = N_EDGES // CHUNK
N_CHUNKS_PAD = 2528
K_CHEB = 6
NC = 2
NS = 16
ROWS_PER_TILE = N_PAD // NS
WB = 128
CPT = N_CHUNKS_PAD // NS
CPT32 = N_CHUNKS_PAD // (NS * NC)

_SC_PARAMS = pltpu.CompilerParams(needs_layout_passes=False,
                                  use_tc_tiling_on_sc=False)


def _mesh():
    return plsc.VectorSubcoreMesh(core_axis_name="c", subcore_axis_name="s")


def _splat(val, i):
    return plsc.load_gather(val, [jnp.full((16,), i, jnp.int32)])


@functools.partial(
    pl.kernel,
    out_type=jax.ShapeDtypeStruct((NC, N_PAD), jnp.float32),
    mesh=_mesh(),
    compiler_params=_SC_PARAMS,
    scratch_types=[
        pltpu.VMEM_SHARED((N_PAD,), jnp.float32),
        pltpu.VMEM((ROWS_PER_TILE,), jnp.float32),
        pltpu.VMEM((1, CHUNK), jnp.int32),
        pltpu.VMEM((CHUNK,), jnp.float32),
        pltpu.VMEM((ROWS_PER_TILE,), jnp.float32),
    ],
)
def _deg_kernel(row2d, lap2d, deg_out, deg_acc, zbuf, ridx, lbuf, dslice):
    cid = lax.axis_index("c")
    sid = lax.axis_index("s")
    r0 = sid * ROWS_PER_TILE

    def _zb(i, _):
        zbuf[pl.ds(i * 16, 16)] = jnp.zeros((16,), jnp.float32)
        return 0
    lax.fori_loop(0, ROWS_PER_TILE // 16, _zb, 0)
    pltpu.sync_copy(zbuf, deg_acc.at[pl.ds(r0, ROWS_PER_TILE)])
    plsc.subcore_barrier()

    wid = sid * NC + cid
    start = wid * CPT32

    def _deg(j, _):
        gj = start + j
        pltpu.sync_copy(row2d.at[gj], ridx.at[0])
        pltpu.sync_copy(lap2d.at[gj], lbuf)
        pltpu.sync_copy(lbuf, deg_acc.at[ridx.at[0]], add=True)
        return 0
    lax.fori_loop(0, CPT32, _deg, 0)
    plsc.subcore_barrier()

    pltpu.sync_copy(deg_acc.at[pl.ds(r0, ROWS_PER_TILE)], dslice)
    pltpu.sync_copy(dslice, deg_out.at[cid, pl.ds(r0, ROWS_PER_TILE)])


def _dis_body(p_ref, o_ref):
    deg = p_ref[0] + p_ref[1]
    y = lax.rsqrt(jnp.maximum(deg, 1e-12))
    o_ref[...] = jnp.where(deg > 0, y, 0.0)


_dis_kernel = pl.pallas_call(
    _dis_body,
    out_shape=jax.ShapeDtypeStruct((N_PAD // 128, 128), jnp.float32),
)


@functools.partial(
    pl.kernel,
    out_type=jax.ShapeDtypeStruct((N_CHUNKS_PAD, CHUNK), jnp.float32),
    mesh=_mesh(),
    compiler_params=_SC_PARAMS,
    scratch_types=[
        pltpu.VMEM((CPT32, CHUNK), jnp.int32),
        pltpu.VMEM((CPT32, CHUNK), jnp.int32),
        pltpu.VMEM((CPT32, CHUNK), jnp.float32),
        pltpu.VMEM((CHUNK,), jnp.float32),
        pltpu.VMEM((N_PAD,), jnp.float32),
    ],
)
def _wn_kernel(row2d, col2d, lap2d, dis, wn2d, rbuf, cbuf, lbuf, wbuf, disbuf):
    cid = lax.axis_index("c")
    sid = lax.axis_index("s")
    pltpu.sync_copy(dis, disbuf)
    wid = sid * NC + cid
    start = wid * CPT32
    pltpu.sync_copy(row2d.at[pl.ds(start, CPT32)], rbuf)
    pltpu.sync_copy(col2d.at[pl.ds(start, CPT32)], cbuf)
    pltpu.sync_copy(lap2d.at[pl.ds(start, CPT32)], lbuf)

    def _wn(j, _):
        for i in range(CHUNK // 16):
            r16 = rbuf[j, pl.ds(i * 16, 16)]
            c16 = cbuf[j, pl.ds(i * 16, 16)]
            dr = plsc.load_gather(disbuf, [r16])
            dc = plsc.load_gather(disbuf, [c16])
            l16 = lbuf[j, pl.ds(i * 16, 16)]
            wbuf[pl.ds(i * 16, 16)] = -(dr * l16 * dc)
        pltpu.sync_copy(wbuf, wn2d.at[start + j])
        return 0
    lax.fori_loop(0, CPT32, _wn, 0)


def _make_prop_kernel(d2):
    nvec = d2 // 16

    @functools.partial(
        pl.kernel,
        out_type=jax.ShapeDtypeStruct((5, NC, N_PAD, d2), jnp.float32),
        mesh=_mesh(),
        compiler_params=_SC_PARAMS,
        scratch_types=[
            pltpu.VMEM_SHARED((N_PAD, d2), jnp.float32),
            pltpu.VMEM((CPT, CHUNK), jnp.int32),
            pltpu.VMEM((CPT, CHUNK), jnp.int32),
            pltpu.VMEM((CHUNK, d2), jnp.float32),
            pltpu.VMEM((CHUNK, d2), jnp.float32),
            pltpu.VMEM((CHUNK,), jnp.float32),
            pltpu.VMEM((CHUNK,), jnp.float32),
            pltpu.VMEM((WB, d2), jnp.float32),
            pltpu.VMEM((WB, d2), jnp.float32),
            pltpu.SemaphoreType.DMA,
            pltpu.SemaphoreType.DMA,
        ],
    )
    def prop_kernel(h2, row2d, col2d, wn2d, zeros, tx,
                    acc, rbuf, cbuf, gbuf0, gbuf1, wbuf0, wbuf1, pbuf, sbuf,
                    gsem0, gsem1):
        cid = lax.axis_index("c")
        sid = lax.axis_index("s")
        r0 = sid * ROWS_PER_TILE
        cstart = sid * CPT

        pltpu.sync_copy(row2d.at[pl.ds(cstart, CPT)], rbuf)
        pltpu.sync_copy(col2d.at[pl.ds(cstart, CPT)], cbuf)
        for q in range(ROWS_PER_TILE // WB):
            pltpu.sync_copy(zeros, acc.at[pl.ds(r0 + q * WB, WB)])
        plsc.subcore_barrier()

        bufs = ((gbuf0, wbuf0, gsem0), (gbuf1, wbuf1, gsem1))

        def _fire(src, j, gb, wb, gs):
            pltpu.async_copy(src.at[cbuf.at[j]], gb, gs)
            pltpu.async_copy(wn2d.at[cstart + j], wb, gs)

        for k in range(1, 6):
            src = h2.at[cid] if k == 1 else tx.at[k - 2, cid]

            for b, (gb, wb, gs) in enumerate(bufs):
                _fire(src, b, gb, wb, gs)

            def _pair(jj, _, src=src):
                j0 = jj * 2
                for b, (gb, wb, gs) in enumerate(bufs):
                    j = j0 + b
                    pltpu.make_async_copy(src.at[cbuf.at[j]], gb, gs).wait()
                    pltpu.make_async_copy(wn2d.at[cstart + j], wb, gs).wait()

                    def _scale(e, _2, gb=gb, wb=wb):
                        w = plsc.load_gather(
                            wb, [jnp.full((16,), e, jnp.int32)])
                        for i in range(nvec):
                            gb[e, pl.ds(i * 16, 16)] = (
                                gb[e, pl.ds(i * 16, 16)] * w)
                        return 0
                    lax.fori_loop(0, CHUNK, _scale, 0, unroll=4)
                    pltpu.sync_copy(gb, acc.at[rbuf.at[j]], add=True)
                    jn = j + 2

                    @pl.when(jn < CPT)
                    def _(gb=gb, wb=wb, gs=gs, jn=jn, src=src):
                        _fire(src, jn, gb, wb, gs)
                return 0
            lax.fori_loop(0, CPT // 2, _pair, 0)
            plsc.subcore_barrier()

            for q in range(ROWS_PER_TILE // WB):
                rb = r0 + q * WB
                pltpu.sync_copy(acc.at[pl.ds(rb, WB)], pbuf)
                if k < 5:
                    pltpu.sync_copy(zeros, acc.at[pl.ds(rb, WB)])
                if k > 1:
                    subsrc = h2.at[cid] if k == 2 else tx.at[k - 3, cid]
                    pltpu.sync_copy(subsrc.at[pl.ds(rb, WB)], sbuf)

                    def _fix(r, _2):
                        for i in range(nvec):
                            pbuf[r, pl.ds(i * 16, 16)] = (
                                2.0 * pbuf[r, pl.ds(i * 16, 16)]
                                - sbuf[r, pl.ds(i * 16, 16)])
                        return 0
                    lax.fori_loop(0, WB, _fix, 0, unroll=4)
                pltpu.sync_copy(pbuf, tx.at[k - 1, cid].at[pl.ds(rb, WB)])
            plsc.subcore_barrier()

    return prop_kernel


def _make_mm_kernel(din, dout):
    d2i, d2o = din // 2, dout // 2
    bn = 1024

    def mm(h_ref, tx_ref, w_ref, b_ref, o_ref):
        acc = jnp.broadcast_to(b_ref[0], (bn, dout))
        for c in range(2):
            acc = acc + jnp.dot(h_ref[c], w_ref[0, c * d2i:(c + 1) * d2i, :],
                                preferred_element_type=jnp.float32)
            for k in range(1, K_CHEB):
                acc = acc + jnp.dot(tx_ref[k - 1, c],
                                    w_ref[k, c * d2i:(c + 1) * d2i, :],
                                    preferred_element_type=jnp.float32)
        acc = jnp.maximum(acc, 0.0)
        for c in range(2):
            o_ref[c] = acc[:, c * d2o:(c + 1) * d2o]

    return pl.pallas_call(
        mm,
        grid=(N_PAD // bn,),
        in_specs=[
            pl.BlockSpec((2, bn, d2i), lambda i: (0, i, 0)),
            pl.BlockSpec((5, 2, bn, d2i), lambda i: (0, 0, i, 0)),
            pl.BlockSpec((K_CHEB, din, dout), lambda i: (0, 0, 0)),
            pl.BlockSpec((1, dout), lambda i: (0, 0)),
        ],
        out_specs=pl.BlockSpec((2, bn, d2o), lambda i: (0, i, 0)),
        out_shape=jax.ShapeDtypeStruct((2, N_PAD, d2o), jnp.float32),
    )


_PROP = {128: _make_prop_kernel(64), 64: _make_prop_kernel(32)}
_MM = {(128, 64): _make_mm_kernel(128, 64), (64, 64): _make_mm_kernel(64, 64),
       (64, 128): _make_mm_kernel(64, 128)}


def kernel(x, edge_index, laplacian, W1, b1, W2, b2, W3, b3, W4, b4):
    pad_c = ((0, N_CHUNKS_PAD - N_CHUNKS), (0, 0))
    row2d = jnp.pad(edge_index[0].reshape(N_CHUNKS, CHUNK), pad_c)
    col2d = jnp.pad(edge_index[1].reshape(N_CHUNKS, CHUNK), pad_c)
    lap2d = jnp.pad(laplacian.reshape(N_CHUNKS, CHUNK), pad_c)

    deg_p = _deg_kernel(row2d, lap2d)
    dis = _dis_kernel(deg_p.reshape(NC, N_PAD // 128, 128)).reshape(N_PAD)
    wn2d = _wn_kernel(row2d, col2d, lap2d, dis)

    xp = jnp.pad(x, ((0, N_PAD - N_NODES), (0, 0)))
    h = xp.reshape(N_PAD, 2, 64).transpose(1, 0, 2)

    w2p = jnp.pad(W2, ((0, 0), (0, 0), (0, 32)))
    b2p = jnp.pad(b2, (0, 32))
    w3p = jnp.pad(W3, ((0, 0), (0, 32), (0, 0)))

    layers = [(128, 64, W1, b1), (64, 64, w2p, b2p),
              (64, 64, w3p, b3), (64, 128, W4, b4)]
    zeros64 = jnp.zeros((WB, 64), jnp.float32)
    zeros32 = jnp.zeros((WB, 32), jnp.float32)
    for din, dout, W, b in layers:
        tx = _PROP[din](h, row2d, col2d, wn2d, zeros64 if din == 128 else zeros32)
        h = _MM[(din, dout)](h, tx, W, b.reshape(1, dout))

    return jnp.concatenate([h[0, :N_NODES], h[1, :N_NODES]], axis=1)
